# Initial kernel scaffold; baseline (speedup 1.0000x reference)
import jax
import jax.numpy as jnp
from jax.experimental import pallas as pl


def kernel(x, node_indices, embedding):
    raise NotImplementedError("write your pallas kernel here")


# speedup vs baseline: 1.2121x; 1.2121x over previous
import jax
import jax.numpy as jnp
from jax import lax
from jax.experimental import pallas as pl
from jax.experimental.pallas import tpu as pltpu
from jax.experimental.pallas import tpu_sc as plsc

N_NODES = 100000
D = 128
B = 50000

L = 16
NSUB = 16

CHUNK = 96

N_COPY = -(-N_NODES // CHUNK)
COPY_LAST = N_NODES - CHUNK
COPY_ITERS = -(-N_COPY // 30)

N_P2 = -(-B // CHUNK)
P2_LAST = B - CHUNK
P2_ITERS = -(-N_P2 // NSUB)

WCHUNK = 2000
N_WCH = B // WCHUNK
GRP = WCHUNK // L


def _body(x_hbm, idx_hbm, emb_hbm, newemb_hbm, pulled_hbm, src_hbm,
          w_v, idxc_v, srcc_v, rows_v, idxb_v, srcb_v, sem):
    c = lax.axis_index("c")
    s = lax.axis_index("s")

    @pl.when(s == 0)
    def _wpass():
        iota = lax.iota(jnp.int32, L)
        for ci in range(N_WCH):
            cstart = ci * WCHUNK
            pltpu.sync_copy(idx_hbm.at[pl.ds(cstart, WCHUNK)], idxc_v)

            def grp(g, carry):
                idxv = idxc_v[pl.ds(g * L, L)]
                jv = (cstart + g * L) + iota
                _, keep = plsc.scan_count(idxv)
                plsc.store_scatter(w_v, [idxv], jv, mask=keep)
                return carry

            lax.fori_loop(0, GRP, grp, 0)
        for ci in range(N_WCH):
            cstart = ci * WCHUNK
            pltpu.sync_copy(idx_hbm.at[pl.ds(cstart, WCHUNK)], idxc_v)

            def grp2(g, carry):
                idxv = idxc_v[pl.ds(g * L, L)]
                srcc_v[pl.ds(g * L, L)] = plsc.load_gather(w_v, [idxv])
                return carry

            lax.fori_loop(0, GRP, grp2, 0)
            pltpu.sync_copy(srcc_v, src_hbm.at[pl.ds(cstart, WCHUNK)])

    @pl.when(s > 0)
    def _copy():
        v = c * (NSUB - 1) + (s - 1)

        def it(i, carry):
            g = jnp.minimum(v + 30 * i, N_COPY - 1)
            start = jnp.minimum(g * CHUNK, COPY_LAST)
            pltpu.sync_copy(emb_hbm.at[pl.ds(start, CHUNK)], rows_v)
            pltpu.sync_copy(rows_v, newemb_hbm.at[pl.ds(start, CHUNK)])
            return carry

        lax.fori_loop(0, COPY_ITERS, it, 0)

    plsc.subcore_barrier()

    def it2(i, carry):
        g = jnp.minimum(s + NSUB * i, N_P2 - 1)
        start = jnp.minimum(g * CHUNK, P2_LAST)
        pltpu.sync_copy(idx_hbm.at[pl.ds(start, CHUNK)], idxb_v)
        pltpu.sync_copy(src_hbm.at[pl.ds(start, CHUNK)], srcb_v)
        pltpu.async_copy(x_hbm.at[srcb_v], rows_v, sem).wait()

        @pl.when((g % 2) == c)
        def _pull():
            pltpu.sync_copy(rows_v, pulled_hbm.at[pl.ds(start, CHUNK)])

        pltpu.async_copy(rows_v, newemb_hbm.at[idxb_v], sem).wait()
        return carry

    lax.fori_loop(0, P2_ITERS, it2, 0)


def kernel(x, node_indices, embedding):
    idx32 = node_indices.astype(jnp.int32)
    f = pl.kernel(
        _body,
        out_type=(
            jax.ShapeDtypeStruct((N_NODES, D), jnp.float32),
            jax.ShapeDtypeStruct((B, D), jnp.float32),
            jax.ShapeDtypeStruct((B,), jnp.int32),
        ),
        mesh=plsc.VectorSubcoreMesh(core_axis_name="c", subcore_axis_name="s"),
        compiler_params=pltpu.CompilerParams(needs_layout_passes=False),
        scratch_types=[
            pltpu.VMEM((N_NODES,), jnp.int32),
            pltpu.VMEM((WCHUNK,), jnp.int32),
            pltpu.VMEM((WCHUNK,), jnp.int32),
            pltpu.VMEM((CHUNK, D), jnp.float32),
            pltpu.VMEM((CHUNK,), jnp.int32),
            pltpu.VMEM((CHUNK,), jnp.int32),
            pltpu.SemaphoreType.DMA,
        ],
    )
    new_emb, pulled, _ = f(x, idx32, embedding)
    return (new_emb, pulled)


# speedup vs baseline: 1.8533x; 1.5290x over previous
import jax
import jax.numpy as jnp
from jax import lax
from jax.experimental import pallas as pl
from jax.experimental.pallas import tpu as pltpu
from jax.experimental.pallas import tpu_sc as plsc

N_NODES = 100000
D = 128
B = 50000

L = 16
NSUB = 16

CHUNK = 80

N_COPY = N_NODES // CHUNK
COPY_LAST = N_NODES - CHUNK
N_CW = 2 * (NSUB - 1)
COPY_ITERS = -(-N_COPY // N_CW)

N_P2 = B // CHUNK
P2_LAST = B - CHUNK
P2_ITERS = -(-N_P2 // NSUB)
P2_QUADS = P2_ITERS // 4

WCHUNK = 2000
N_WCH = B // WCHUNK
GRP = WCHUNK // L


def _body(x_hbm, idx_hbm, emb_hbm, newemb_hbm, pulled_hbm, src_hbm,
          w_v, idxc0, idxc1, srcc0, srcc1, rows0, rows1,
          ib0, ib1, ib2, ib3, sb0, sb1, sb2, sb3,
          sio0, sio1, sio2, sio3, sg0, sg1, ssc0, ssc1, sp0, sp1):
    c = lax.axis_index("c")
    s = lax.axis_index("s")
    idxc = (idxc0, idxc1)
    srcc = (srcc0, srcc1)
    rows = (rows0, rows1)
    ib = (ib0, ib1, ib2, ib3)
    sb = (sb0, sb1, sb2, sb3)
    sio = (sio0, sio1, sio2, sio3)
    sg = (sg0, sg1)
    ssc = (ssc0, ssc1)
    sp = (sp0, sp1)

    @pl.when(s == 0)
    def _wpass():
        iota = lax.iota(jnp.int32, L)
        pltpu.async_copy(idx_hbm.at[pl.ds(0, WCHUNK)], idxc[0], sio[0])
        for ci in range(N_WCH):
            b = ci % 2
            if ci + 1 < N_WCH:
                pltpu.async_copy(
                    idx_hbm.at[pl.ds((ci + 1) * WCHUNK, WCHUNK)],
                    idxc[1 - b], sio[1 - b])
            pltpu.make_async_copy(
                idx_hbm.at[pl.ds(ci * WCHUNK, WCHUNK)], idxc[b], sio[b]).wait()
            cstart = ci * WCHUNK

            def grp(g, carry, _b=b, _cstart=cstart):
                idxv = idxc[_b][pl.ds(g * L, L)]
                jv = (_cstart + g * L) + iota
                _, keep = plsc.scan_count(idxv)
                plsc.store_scatter(w_v, [idxv], jv, mask=keep)
                return carry

            lax.fori_loop(0, GRP, grp, 0)
        pltpu.async_copy(idx_hbm.at[pl.ds(0, WCHUNK)], idxc[0], sio[0])
        for ci in range(N_WCH):
            b = ci % 2
            if ci + 1 < N_WCH:
                pltpu.async_copy(
                    idx_hbm.at[pl.ds((ci + 1) * WCHUNK, WCHUNK)],
                    idxc[1 - b], sio[1 - b])
            pltpu.make_async_copy(
                idx_hbm.at[pl.ds(ci * WCHUNK, WCHUNK)], idxc[b], sio[b]).wait()
            if ci >= 2:
                pltpu.make_async_copy(
                    srcc[b], src_hbm.at[pl.ds((ci - 2) * WCHUNK, WCHUNK)],
                    ssc[b]).wait()

            def grp2(g, carry, _b=b):
                idxv = idxc[_b][pl.ds(g * L, L)]
                srcc[_b][pl.ds(g * L, L)] = plsc.load_gather(w_v, [idxv])
                return carry

            lax.fori_loop(0, GRP, grp2, 0)
            pltpu.async_copy(
                srcc[b], src_hbm.at[pl.ds(ci * WCHUNK, WCHUNK)], ssc[b])
        for ci in (N_WCH - 2, N_WCH - 1):
            pltpu.make_async_copy(
                srcc[ci % 2], src_hbm.at[pl.ds(ci * WCHUNK, WCHUNK)],
                ssc[ci % 2]).wait()

    @pl.when(s > 0)
    def _copy():
        v = c * (NSUB - 1) + (s - 1)

        def cstart_of(j):
            g = jnp.minimum(v + N_CW * j, N_COPY - 1)
            return jnp.minimum(g * CHUNK, COPY_LAST)

        pltpu.async_copy(emb_hbm.at[pl.ds(cstart_of(0), CHUNK)], rows[0], sg[0])

        def it(jp, carry):
            for bb in range(2):
                j = jp * 2 + bb
                st = cstart_of(j)

                @pl.when(j + 1 < COPY_ITERS)
                def _prefetch():
                    @pl.when(j >= 1)
                    def _drain_out():
                        pltpu.make_async_copy(
                            rows[1 - bb],
                            newemb_hbm.at[pl.ds(cstart_of(j - 1), CHUNK)],
                            ssc[1 - bb]).wait()
                    pltpu.async_copy(
                        emb_hbm.at[pl.ds(cstart_of(j + 1), CHUNK)],
                        rows[1 - bb], sg[1 - bb])

                pltpu.make_async_copy(
                    emb_hbm.at[pl.ds(st, CHUNK)], rows[bb], sg[bb]).wait()
                pltpu.async_copy(
                    rows[bb], newemb_hbm.at[pl.ds(st, CHUNK)], ssc[bb])
            return carry

        lax.fori_loop(0, COPY_ITERS // 2, it, 0)
        for jz in (COPY_ITERS - 2, COPY_ITERS - 1):
            pltpu.make_async_copy(
                rows[jz % 2], newemb_hbm.at[pl.ds(cstart_of(jz), CHUNK)],
                ssc[jz % 2]).wait()

    plsc.subcore_barrier()

    def p2start_of(i):
        g = jnp.minimum(s + NSUB * i, N_P2 - 1)
        return g, jnp.minimum(g * CHUNK, P2_LAST)

    _, st0 = p2start_of(0)
    pltpu.async_copy(idx_hbm.at[pl.ds(st0, CHUNK)], ib[0], sio[0])
    pltpu.async_copy(src_hbm.at[pl.ds(st0, CHUNK)], sb[0], sio[0])

    def it2(q, carry):
        for bb in range(4):
            i = q * 4 + bb
            b2 = bb % 2
            g, st = p2start_of(i)

            @pl.when(i + 1 < P2_ITERS)
            def _prefetch():
                _, st1 = p2start_of(i + 1)
                pltpu.async_copy(
                    idx_hbm.at[pl.ds(st1, CHUNK)], ib[(bb + 1) % 4],
                    sio[(bb + 1) % 4])
                pltpu.async_copy(
                    src_hbm.at[pl.ds(st1, CHUNK)], sb[(bb + 1) % 4],
                    sio[(bb + 1) % 4])

            pltpu.make_async_copy(
                idx_hbm.at[pl.ds(st, CHUNK)], ib[bb], sio[bb]).wait()
            pltpu.make_async_copy(
                src_hbm.at[pl.ds(st, CHUNK)], sb[bb], sio[bb]).wait()

            @pl.when(i >= 2)
            def _drain_prev():
                g2, st2 = p2start_of(i - 2)
                pltpu.make_async_copy(
                    rows[b2], newemb_hbm.at[ib[(bb + 2) % 4]], ssc[b2]).wait()

                @pl.when((g2 % 2) == c)
                def _drain_pull():
                    pltpu.make_async_copy(
                        rows[b2], pulled_hbm.at[pl.ds(st2, CHUNK)],
                        sp[b2]).wait()

            pltpu.async_copy(x_hbm.at[sb[bb]], rows[b2], sg[b2]).wait()

            @pl.when((g % 2) == c)
            def _pull():
                pltpu.async_copy(
                    rows[b2], pulled_hbm.at[pl.ds(st, CHUNK)], sp[b2])

            pltpu.async_copy(rows[b2], newemb_hbm.at[ib[bb]], ssc[b2])
        return carry

    lax.fori_loop(0, P2_QUADS, it2, 0)
    for iz in (P2_ITERS - 2, P2_ITERS - 1):
        b2 = iz % 2
        gz, stz = p2start_of(iz)
        pltpu.make_async_copy(
            rows[b2], newemb_hbm.at[ib[iz % 4]], ssc[b2]).wait()

        @pl.when((gz % 2) == c)
        def _drain_pull_z():
            pltpu.make_async_copy(
                rows[b2], pulled_hbm.at[pl.ds(stz, CHUNK)], sp[b2]).wait()


def kernel(x, node_indices, embedding):
    idx32 = node_indices.astype(jnp.int32)
    f = pl.kernel(
        _body,
        out_type=(
            jax.ShapeDtypeStruct((N_NODES, D), jnp.float32),
            jax.ShapeDtypeStruct((B, D), jnp.float32),
            jax.ShapeDtypeStruct((B,), jnp.int32),
        ),
        mesh=plsc.VectorSubcoreMesh(core_axis_name="c", subcore_axis_name="s"),
        compiler_params=pltpu.CompilerParams(needs_layout_passes=False),
        scratch_types=[
            pltpu.VMEM((N_NODES,), jnp.int32),
            pltpu.VMEM((WCHUNK,), jnp.int32),
            pltpu.VMEM((WCHUNK,), jnp.int32),
            pltpu.VMEM((WCHUNK,), jnp.int32),
            pltpu.VMEM((WCHUNK,), jnp.int32),
            pltpu.VMEM((CHUNK, D), jnp.float32),
            pltpu.VMEM((CHUNK, D), jnp.float32),
            pltpu.VMEM((CHUNK,), jnp.int32),
            pltpu.VMEM((CHUNK,), jnp.int32),
            pltpu.VMEM((CHUNK,), jnp.int32),
            pltpu.VMEM((CHUNK,), jnp.int32),
            pltpu.VMEM((CHUNK,), jnp.int32),
            pltpu.VMEM((CHUNK,), jnp.int32),
            pltpu.VMEM((CHUNK,), jnp.int32),
            pltpu.VMEM((CHUNK,), jnp.int32),
            pltpu.SemaphoreType.DMA,
            pltpu.SemaphoreType.DMA,
            pltpu.SemaphoreType.DMA,
            pltpu.SemaphoreType.DMA,
            pltpu.SemaphoreType.DMA,
            pltpu.SemaphoreType.DMA,
            pltpu.SemaphoreType.DMA,
            pltpu.SemaphoreType.DMA,
            pltpu.SemaphoreType.DMA,
            pltpu.SemaphoreType.DMA,
        ],
    )
    new_emb, pulled, _ = f(x, idx32, embedding)
    return (new_emb, pulled)
